# baseline (speedup 1.0000x reference)
import functools

import jax
import jax.numpy as jnp
import numpy as np
from jax import lax
from jax.experimental import pallas as pl
from jax.experimental.pallas import tpu as pltpu
from jax.experimental.pallas import tpu_sc as plsc

INPUT_SIZE = 1024
HIDDEN_SIZE = 4096
NUM_CLASSES = 16384
K = 10
L = 8
NUM_SAMPLED = 1024
BATCH = 2048

NB = 1 << K
NBF = L * NB
SC_CORES = 2
SC_SUBCORES = 16
QPW = BATCH * L // (SC_CORES * SC_SUBCORES)
HPW = HIDDEN_SIZE // SC_SUBCORES

_G_NP = np.zeros((L * K, L), dtype=np.float32)
for _l in range(L):
    for _k in range(K):
        _G_NP[_l * K + _k, _l] = float(2 ** _k)


def _hash_kernel(x_ref, w1_ref, hwt_ref, g_ref, cq_ref, cw_ref):
    proj_q = jnp.dot(x_ref[...], hwt_ref[...],
                     preferred_element_type=jnp.float32)
    proj_w = jnp.dot(w1_ref[...], hwt_ref[...],
                     preferred_element_type=jnp.float32)
    g = g_ref[...]
    codes_q = jnp.dot((proj_q > 0).astype(jnp.float32), g,
                      preferred_element_type=jnp.float32,
                      precision=jax.lax.Precision.HIGHEST).astype(jnp.int32)
    codes_w = jnp.dot((proj_w > 0).astype(jnp.float32), g,
                      preferred_element_type=jnp.float32,
                      precision=jax.lax.Precision.HIGHEST).astype(jnp.int32)
    cq_ref[...] = codes_q + NB * jax.lax.broadcasted_iota(
        jnp.int32, (BATCH, L), 1)
    cw_ref[...] = codes_w.T + NB * jax.lax.broadcasted_iota(
        jnp.int32, (L, HIDDEN_SIZE), 0)


def _sc_counts_body(cq_ref, cw_ref, out_ref,
                    qbuf, ones_v, zbuf, wbuf, gbuf, cnt_v, shared_hist):
    ci = lax.axis_index("c")
    si = lax.axis_index("s")
    for i in range(NBF // SC_SUBCORES // 16):
        zbuf[pl.ds(i * 16, 16)] = jnp.zeros((16,), jnp.int32)
    pltpu.sync_copy(zbuf, shared_hist.at[pl.ds(si * (NBF // SC_SUBCORES),
                                               NBF // SC_SUBCORES)])
    plsc.subcore_barrier()
    base = ci * (BATCH * L // SC_CORES) + si * QPW
    for j in range(QPW // 128):
        pltpu.sync_copy(cq_ref.at[pl.ds(base + j * 128, 128)], qbuf.at[j])
    for i in range(128 // 16):
        ones_v[pl.ds(i * 16, 16)] = jnp.ones((16,), jnp.int32)
    for j in range(QPW // 128):
        pltpu.sync_copy(ones_v, shared_hist.at[qbuf.at[j]], add=True)
    plsc.subcore_barrier()
    NR = L * HPW // 128
    for r in range(NR):
        pltpu.sync_copy(cw_ref.at[pl.ds(si * L * HPW + r * 128, 128)],
                        wbuf.at[r])
    for r in range(NR):
        pltpu.sync_copy(shared_hist.at[wbuf.at[r]], gbuf.at[r])
    for jh in range(HPW // 16):
        acc = jnp.zeros((16,), jnp.int32)
        for l in range(L):
            p = l * HPW + jh * 16
            acc = acc + gbuf[p // 128, pl.ds(p % 128, 16)]
        cnt_v[pl.ds(jh * 16, 16)] = acc
    pltpu.sync_copy(cnt_v, out_ref.at[pl.ds(ci * HIDDEN_SIZE + si * HPW, HPW)])


def _make_sc_counts():
    return functools.partial(
        pl.kernel,
        out_type=jax.ShapeDtypeStruct((SC_CORES * HIDDEN_SIZE,), jnp.int32),
        mesh=plsc.VectorSubcoreMesh(core_axis_name="c", subcore_axis_name="s",
                                    num_cores=SC_CORES,
                                    num_subcores=SC_SUBCORES),
        scratch_types=[
            pltpu.VMEM((QPW // 128, 128), jnp.int32),
            pltpu.VMEM((128,), jnp.int32),
            pltpu.VMEM((NBF // SC_SUBCORES,), jnp.int32),
            pltpu.VMEM((L * HPW // 128, 128), jnp.int32),
            pltpu.VMEM((L * HPW // 128, 128), jnp.int32),
            pltpu.VMEM((HPW,), jnp.int32),
            pltpu.VMEM_SHARED((NBF,), jnp.int32),
        ],
    )(_sc_counts_body)


def _selt_kernel(c2_ref, sel_ref):
    counts = jnp.sum(c2_ref[...], axis=0, keepdims=True)
    hidx = jax.lax.broadcasted_iota(jnp.int32, (1, HIDDEN_SIZE), 1)
    keys = counts * HIDDEN_SIZE + (HIDDEN_SIZE - 1 - hidx)
    def body(_, lohi):
        lo, hi = lohi
        mid = (lo + hi) >> 1
        cnt = jnp.sum((keys >= mid).astype(jnp.int32))
        ok = cnt >= NUM_SAMPLED
        return (jnp.where(ok, mid, lo), jnp.where(ok, hi, mid))
    lo, _ = jax.lax.fori_loop(0, 27, body, (jnp.int32(0), jnp.int32(1 << 27)))
    mask_row = keys >= lo
    mask_bf = mask_row.astype(jnp.bfloat16)
    CH = 1024
    rank_chunks = []
    for j in range(HIDDEN_SIZE // CH):
        hp = jax.lax.broadcasted_iota(jnp.int32, (HIDDEN_SIZE, CH), 0)
        dst = jax.lax.broadcasted_iota(jnp.int32, (HIDDEN_SIZE, CH), 1)
        tri = (hp < (dst + j * CH)).astype(jnp.bfloat16)
        rank_chunks.append(jnp.dot(mask_bf, tri,
                                   preferred_element_type=jnp.float32))
    rank_i = jnp.concatenate(rank_chunks, axis=1).astype(jnp.int32)
    sidx = jax.lax.broadcasted_iota(jnp.int32, (NUM_SAMPLED, HIDDEN_SIZE), 0)
    sel_t = (rank_i == sidx) & mask_row
    sel_ref[...] = sel_t.astype(jnp.bfloat16)


def _hidden_kernel(x_ref, w1_ref, b1_ref, sel_ref, out_ref):
    w1 = w1_ref[...].astype(jnp.bfloat16)
    sel = sel_ref[...]
    w1s = jax.lax.dot_general(sel, w1, (((1,), (0,)), ((), ())),
                              preferred_element_type=jnp.float32)
    w1s = w1s.astype(jnp.bfloat16)
    b1s = jax.lax.dot_general(b1_ref[...].astype(jnp.bfloat16), sel,
                              (((1,), (1,)), ((), ())),
                              preferred_element_type=jnp.float32)
    x = x_ref[...].astype(jnp.bfloat16)
    acc = jax.lax.dot_general(x, w1s, (((1,), (1,)), ((), ())),
                              preferred_element_type=jnp.float32)
    out_ref[...] = jnp.maximum(acc + b1s, 0.0).astype(jnp.bfloat16)


def _out_kernel(h_ref, w2_ref, sel_ref, b2_ref, out_ref):
    w2 = w2_ref[...].astype(jnp.bfloat16)
    w2s = jax.lax.dot_general(w2, sel_ref[...], (((1,), (1,)), ((), ())),
                              preferred_element_type=jnp.float32)
    w2s = w2s.astype(jnp.bfloat16)
    acc = jax.lax.dot_general(h_ref[...], w2s, (((1,), (1,)), ((), ())),
                              preferred_element_type=jnp.float32)
    out_ref[...] = acc + b2_ref[...]


@jax.jit
def kernel(X, W1, b1, Hw, W2, b2):
    hw_t = Hw.reshape(L * K, INPUT_SIZE).T
    g = jnp.asarray(_G_NP)

    cq, cw = pl.pallas_call(
        _hash_kernel,
        out_shape=[jax.ShapeDtypeStruct((BATCH, L), jnp.int32),
                   jax.ShapeDtypeStruct((L, HIDDEN_SIZE), jnp.int32)],
    )(X, W1, hw_t, g)

    cw_tiled = cw.reshape(L, SC_SUBCORES, HPW).transpose(1, 0, 2).reshape(-1)
    counts2 = _make_sc_counts()(cq.reshape(BATCH * L), cw_tiled)

    sel = pl.pallas_call(
        _selt_kernel,
        out_shape=jax.ShapeDtypeStruct((NUM_SAMPLED, HIDDEN_SIZE), jnp.bfloat16),
    )(counts2.reshape(SC_CORES, HIDDEN_SIZE))

    b1r = b1.reshape(1, HIDDEN_SIZE)
    hlog_s = pl.pallas_call(
        _hidden_kernel,
        out_shape=jax.ShapeDtypeStruct((BATCH, NUM_SAMPLED), jnp.bfloat16),
    )(X, W1, b1r, sel)

    b2r = b2.reshape(1, NUM_CLASSES)
    CB = 512
    out = pl.pallas_call(
        _out_kernel,
        grid=(NUM_CLASSES // CB,),
        in_specs=[
            pl.BlockSpec((BATCH, NUM_SAMPLED), lambda i: (0, 0)),
            pl.BlockSpec((CB, HIDDEN_SIZE), lambda i: (i, 0)),
            pl.BlockSpec((NUM_SAMPLED, HIDDEN_SIZE), lambda i: (0, 0)),
            pl.BlockSpec((1, CB), lambda i: (0, i)),
        ],
        out_specs=pl.BlockSpec((BATCH, CB), lambda i: (0, i)),
        out_shape=jax.ShapeDtypeStruct((BATCH, NUM_CLASSES), jnp.float32),
    )(hlog_s, W2, sel, b2r)
    return out


# speedup vs baseline: 1.0341x; 1.0341x over previous
import functools

import jax
import jax.numpy as jnp
import numpy as np
from jax import lax
from jax.experimental import pallas as pl
from jax.experimental.pallas import tpu as pltpu
from jax.experimental.pallas import tpu_sc as plsc

INPUT_SIZE = 1024
HIDDEN_SIZE = 4096
NUM_CLASSES = 16384
K = 10
L = 8
NUM_SAMPLED = 1024
BATCH = 2048

NB = 1 << K
NBF = L * NB
SC_CORES = 2
SC_SUBCORES = 16
QPW = BATCH * L // (SC_CORES * SC_SUBCORES)
HPW = HIDDEN_SIZE // SC_SUBCORES

_G_NP = np.zeros((L * K, L), dtype=np.float32)
for _l in range(L):
    for _k in range(K):
        _G_NP[_l * K + _k, _l] = float(2 ** _k)


def _hash_kernel(x_ref, w1_ref, hwt_ref, g_ref, cq_ref, cw_ref):
    proj_q = jnp.dot(x_ref[...], hwt_ref[...],
                     preferred_element_type=jnp.float32)
    proj_w = jnp.dot(w1_ref[...], hwt_ref[...],
                     preferred_element_type=jnp.float32)
    g = g_ref[...]
    codes_q = jnp.dot((proj_q > 0).astype(jnp.float32), g,
                      preferred_element_type=jnp.float32,
                      precision=jax.lax.Precision.HIGHEST).astype(jnp.int32)
    codes_w = jnp.dot((proj_w > 0).astype(jnp.float32), g,
                      preferred_element_type=jnp.float32,
                      precision=jax.lax.Precision.HIGHEST).astype(jnp.int32)
    cq_ref[...] = codes_q + NB * jax.lax.broadcasted_iota(
        jnp.int32, (BATCH, L), 1)
    cw_ref[...] = codes_w.T + NB * jax.lax.broadcasted_iota(
        jnp.int32, (L, HIDDEN_SIZE), 0)


def _sc_counts_body(cq_ref, cw_ref, out_ref,
                    qbuf, ones_v, zbuf, wbuf, gbuf, cnt_v, shared_hist):
    ci = lax.axis_index("c")
    si = lax.axis_index("s")
    for i in range(NBF // SC_SUBCORES // 16):
        zbuf[pl.ds(i * 16, 16)] = jnp.zeros((16,), jnp.int32)
    pltpu.sync_copy(zbuf, shared_hist.at[pl.ds(si * (NBF // SC_SUBCORES),
                                               NBF // SC_SUBCORES)])
    plsc.subcore_barrier()
    base = ci * (BATCH * L // SC_CORES) + si * QPW
    for j in range(QPW // 128):
        pltpu.sync_copy(cq_ref.at[pl.ds(base + j * 128, 128)], qbuf.at[j])
    for i in range(128 // 16):
        ones_v[pl.ds(i * 16, 16)] = jnp.ones((16,), jnp.int32)
    for j in range(QPW // 128):
        pltpu.sync_copy(ones_v, shared_hist.at[qbuf.at[j]], add=True)
    plsc.subcore_barrier()
    NR = L * HPW // 128
    for r in range(NR):
        pltpu.sync_copy(cw_ref.at[pl.ds(si * L * HPW + r * 128, 128)],
                        wbuf.at[r])
    for r in range(NR):
        pltpu.sync_copy(shared_hist.at[wbuf.at[r]], gbuf.at[r])
    for jh in range(HPW // 16):
        acc = jnp.zeros((16,), jnp.int32)
        for l in range(L):
            p = l * HPW + jh * 16
            acc = acc + gbuf[p // 128, pl.ds(p % 128, 16)]
        cnt_v[pl.ds(jh * 16, 16)] = acc
    pltpu.sync_copy(cnt_v, out_ref.at[pl.ds(ci * HIDDEN_SIZE + si * HPW, HPW)])


def _make_sc_counts():
    return functools.partial(
        pl.kernel,
        out_type=jax.ShapeDtypeStruct((SC_CORES * HIDDEN_SIZE,), jnp.int32),
        mesh=plsc.VectorSubcoreMesh(core_axis_name="c", subcore_axis_name="s",
                                    num_cores=SC_CORES,
                                    num_subcores=SC_SUBCORES),
        scratch_types=[
            pltpu.VMEM((QPW // 128, 128), jnp.int32),
            pltpu.VMEM((128,), jnp.int32),
            pltpu.VMEM((NBF // SC_SUBCORES,), jnp.int32),
            pltpu.VMEM((L * HPW // 128, 128), jnp.int32),
            pltpu.VMEM((L * HPW // 128, 128), jnp.int32),
            pltpu.VMEM((HPW,), jnp.int32),
            pltpu.VMEM_SHARED((NBF,), jnp.int32),
        ],
    )(_sc_counts_body)


def _hidden_kernel(c2_ref, x_ref, w1_ref, b1_ref, out_ref, sel_ref):
    counts = jnp.sum(c2_ref[...], axis=0, keepdims=True)
    hidx = jax.lax.broadcasted_iota(jnp.int32, (1, HIDDEN_SIZE), 1)
    keys = counts * HIDDEN_SIZE + (HIDDEN_SIZE - 1 - hidx)
    def body(_, lohi):
        lo, hi = lohi
        mid = (lo + hi) >> 1
        cnt = jnp.sum((keys >= mid).astype(jnp.int32))
        ok = cnt >= NUM_SAMPLED
        return (jnp.where(ok, mid, lo), jnp.where(ok, hi, mid))
    lo, _ = jax.lax.fori_loop(0, 27, body, (jnp.int32(0), jnp.int32(1 << 27)))
    mask_row = keys >= lo
    mask_bf = mask_row.astype(jnp.bfloat16)
    CH = 1024
    rank_chunks = []
    for j in range(HIDDEN_SIZE // CH):
        hp = jax.lax.broadcasted_iota(jnp.int32, (HIDDEN_SIZE, CH), 0)
        dst = jax.lax.broadcasted_iota(jnp.int32, (HIDDEN_SIZE, CH), 1)
        tri = (hp < (dst + j * CH)).astype(jnp.bfloat16)
        rank_chunks.append(jnp.dot(mask_bf, tri,
                                   preferred_element_type=jnp.float32))
    rank_i = jnp.concatenate(rank_chunks, axis=1).astype(jnp.int32)
    sidx = jax.lax.broadcasted_iota(jnp.int32, (NUM_SAMPLED, HIDDEN_SIZE), 0)
    sel = ((rank_i == sidx) & mask_row).astype(jnp.bfloat16)
    sel_ref[...] = sel
    w1 = w1_ref[...].astype(jnp.bfloat16)
    w1s = jax.lax.dot_general(sel, w1, (((1,), (0,)), ((), ())),
                              preferred_element_type=jnp.float32)
    w1s = w1s.astype(jnp.bfloat16)
    b1s = jax.lax.dot_general(b1_ref[...].astype(jnp.bfloat16), sel,
                              (((1,), (1,)), ((), ())),
                              preferred_element_type=jnp.float32)
    x = x_ref[...].astype(jnp.bfloat16)
    acc = jax.lax.dot_general(x, w1s, (((1,), (1,)), ((), ())),
                              preferred_element_type=jnp.float32)
    out_ref[...] = jnp.maximum(acc + b1s, 0.0).astype(jnp.bfloat16)


def _out_kernel(h_ref, w2_ref, sel_ref, b2_ref, out_ref):
    w2 = w2_ref[...].astype(jnp.bfloat16)
    w2s = jax.lax.dot_general(w2, sel_ref[...], (((1,), (1,)), ((), ())),
                              preferred_element_type=jnp.float32)
    w2s = w2s.astype(jnp.bfloat16)
    acc = jax.lax.dot_general(h_ref[...], w2s, (((1,), (1,)), ((), ())),
                              preferred_element_type=jnp.float32)
    out_ref[...] = acc + b2_ref[...]


@jax.jit
def kernel(X, W1, b1, Hw, W2, b2):
    hw_t = Hw.reshape(L * K, INPUT_SIZE).T
    g = jnp.asarray(_G_NP)

    cq, cw = pl.pallas_call(
        _hash_kernel,
        out_shape=[jax.ShapeDtypeStruct((BATCH, L), jnp.int32),
                   jax.ShapeDtypeStruct((L, HIDDEN_SIZE), jnp.int32)],
    )(X, W1, hw_t, g)

    cw_tiled = cw.reshape(L, SC_SUBCORES, HPW).transpose(1, 0, 2).reshape(-1)
    counts2 = _make_sc_counts()(cq.reshape(BATCH * L), cw_tiled)

    b1r = b1.reshape(1, HIDDEN_SIZE)
    hlog_s, sel = pl.pallas_call(
        _hidden_kernel,
        out_shape=[jax.ShapeDtypeStruct((BATCH, NUM_SAMPLED), jnp.bfloat16),
                   jax.ShapeDtypeStruct((NUM_SAMPLED, HIDDEN_SIZE), jnp.bfloat16)],
    )(counts2.reshape(SC_CORES, HIDDEN_SIZE), X, W1, b1r)

    b2r = b2.reshape(1, NUM_CLASSES)
    CB = 512
    out = pl.pallas_call(
        _out_kernel,
        grid=(NUM_CLASSES // CB,),
        in_specs=[
            pl.BlockSpec((BATCH, NUM_SAMPLED), lambda i: (0, 0)),
            pl.BlockSpec((CB, HIDDEN_SIZE), lambda i: (i, 0)),
            pl.BlockSpec((NUM_SAMPLED, HIDDEN_SIZE), lambda i: (0, 0)),
            pl.BlockSpec((1, CB), lambda i: (0, i)),
        ],
        out_specs=pl.BlockSpec((BATCH, CB), lambda i: (0, i)),
        out_shape=jax.ShapeDtypeStruct((BATCH, NUM_CLASSES), jnp.float32),
    )(hlog_s, W2, sel, b2r)
    return out


# speedup vs baseline: 1.0665x; 1.0313x over previous
import functools

import jax
import jax.numpy as jnp
import numpy as np
from jax import lax
from jax.experimental import pallas as pl
from jax.experimental.pallas import tpu as pltpu
from jax.experimental.pallas import tpu_sc as plsc

INPUT_SIZE = 1024
HIDDEN_SIZE = 4096
NUM_CLASSES = 16384
K = 10
L = 8
NUM_SAMPLED = 1024
BATCH = 2048

NB = 1 << K
NBF = L * NB
SC_CORES = 2
SC_SUBCORES = 16
QPW = BATCH * L // (SC_CORES * SC_SUBCORES)
HPW = HIDDEN_SIZE // SC_SUBCORES

_G_NP = np.zeros((L * K, L), dtype=np.float32)
for _l in range(L):
    for _k in range(K):
        _G_NP[_l * K + _k, _l] = float(2 ** _k)


def _hash_kernel(x_ref, w1_ref, hwt_ref, g_ref, cq_ref, cw_ref):
    proj_q = jnp.dot(x_ref[...], hwt_ref[...],
                     preferred_element_type=jnp.float32)
    proj_w = jnp.dot(w1_ref[...], hwt_ref[...],
                     preferred_element_type=jnp.float32)
    g = g_ref[...]
    codes_q = jnp.dot((proj_q > 0).astype(jnp.float32), g,
                      preferred_element_type=jnp.float32,
                      precision=jax.lax.Precision.HIGHEST).astype(jnp.int32)
    codes_w = jnp.dot((proj_w > 0).astype(jnp.float32), g,
                      preferred_element_type=jnp.float32,
                      precision=jax.lax.Precision.HIGHEST).astype(jnp.int32)
    cq_ref[...] = codes_q + NB * jax.lax.broadcasted_iota(
        jnp.int32, (BATCH, L), 1)
    cw_ref[...] = codes_w.T + NB * jax.lax.broadcasted_iota(
        jnp.int32, (L, HIDDEN_SIZE), 0)


def _sc_counts_body(cq_ref, cw_ref, out_ref,
                    qbuf, ones_v, zbuf, wbuf, gbuf, cnt_v, shared_hist, sem):
    ci = lax.axis_index("c")
    si = lax.axis_index("s")
    for i in range(NBF // SC_SUBCORES // 16):
        zbuf[pl.ds(i * 16, 16)] = jnp.zeros((16,), jnp.int32)
    pltpu.sync_copy(zbuf, shared_hist.at[pl.ds(si * (NBF // SC_SUBCORES),
                                               NBF // SC_SUBCORES)])
    plsc.subcore_barrier()
    base = ci * (BATCH * L // SC_CORES) + si * QPW
    qcopies = [pltpu.async_copy(cq_ref.at[pl.ds(base + j * 128, 128)],
                                qbuf.at[j], sem)
               for j in range(QPW // 128)]
    for i in range(128 // 16):
        ones_v[pl.ds(i * 16, 16)] = jnp.ones((16,), jnp.int32)
    for c in qcopies:
        c.wait()
    for j in range(QPW // 128):
        pltpu.sync_copy(ones_v, shared_hist.at[qbuf.at[j]], add=True)
    plsc.subcore_barrier()
    NR = L * HPW // 128
    wcopies = [pltpu.async_copy(cw_ref.at[pl.ds(si * L * HPW + r * 128, 128)],
                                wbuf.at[r], sem)
               for r in range(NR)]
    for c in wcopies:
        c.wait()
    gcopies = [pltpu.async_copy(shared_hist.at[wbuf.at[r]], gbuf.at[r], sem)
               for r in range(NR)]
    for c in gcopies:
        c.wait()
    for jh in range(HPW // 16):
        acc = jnp.zeros((16,), jnp.int32)
        for l in range(L):
            p = l * HPW + jh * 16
            acc = acc + gbuf[p // 128, pl.ds(p % 128, 16)]
        cnt_v[pl.ds(jh * 16, 16)] = acc
    pltpu.sync_copy(cnt_v, out_ref.at[pl.ds(ci * HIDDEN_SIZE + si * HPW, HPW)])


def _make_sc_counts():
    return functools.partial(
        pl.kernel,
        out_type=jax.ShapeDtypeStruct((SC_CORES * HIDDEN_SIZE,), jnp.int32),
        mesh=plsc.VectorSubcoreMesh(core_axis_name="c", subcore_axis_name="s",
                                    num_cores=SC_CORES,
                                    num_subcores=SC_SUBCORES),
        scratch_types=[
            pltpu.VMEM((QPW // 128, 128), jnp.int32),
            pltpu.VMEM((128,), jnp.int32),
            pltpu.VMEM((NBF // SC_SUBCORES,), jnp.int32),
            pltpu.VMEM((L * HPW // 128, 128), jnp.int32),
            pltpu.VMEM((L * HPW // 128, 128), jnp.int32),
            pltpu.VMEM((HPW,), jnp.int32),
            pltpu.VMEM_SHARED((NBF,), jnp.int32),
            pltpu.SemaphoreType.DMA,
        ],
    )(_sc_counts_body)


def _hidden_kernel(c2_ref, x_ref, w1_ref, b1_ref, out_ref, sel_ref):
    counts = jnp.sum(c2_ref[...], axis=0, keepdims=True)
    hidx = jax.lax.broadcasted_iota(jnp.int32, (1, HIDDEN_SIZE), 1)
    keys = counts * HIDDEN_SIZE + (HIDDEN_SIZE - 1 - hidx)
    def body(_, lohi):
        lo, hi = lohi
        mid = (lo + hi) >> 1
        cnt = jnp.sum((keys >= mid).astype(jnp.int32))
        ok = cnt >= NUM_SAMPLED
        return (jnp.where(ok, mid, lo), jnp.where(ok, hi, mid))
    lo, _ = jax.lax.fori_loop(0, 27, body, (jnp.int32(0), jnp.int32(1 << 27)))
    mask_row = keys >= lo
    mask_bf = mask_row.astype(jnp.bfloat16)
    CH = 1024
    rank_chunks = []
    for j in range(HIDDEN_SIZE // CH):
        hp = jax.lax.broadcasted_iota(jnp.int32, (HIDDEN_SIZE, CH), 0)
        dst = jax.lax.broadcasted_iota(jnp.int32, (HIDDEN_SIZE, CH), 1)
        tri = (hp < (dst + j * CH)).astype(jnp.bfloat16)
        rank_chunks.append(jnp.dot(mask_bf, tri,
                                   preferred_element_type=jnp.float32))
    rank_i = jnp.concatenate(rank_chunks, axis=1).astype(jnp.int32)
    sidx = jax.lax.broadcasted_iota(jnp.int32, (NUM_SAMPLED, HIDDEN_SIZE), 0)
    sel = ((rank_i == sidx) & mask_row).astype(jnp.bfloat16)
    sel_ref[...] = sel
    w1 = w1_ref[...].astype(jnp.bfloat16)
    w1s = jax.lax.dot_general(sel, w1, (((1,), (0,)), ((), ())),
                              preferred_element_type=jnp.float32)
    w1s = w1s.astype(jnp.bfloat16)
    b1s = jax.lax.dot_general(b1_ref[...].astype(jnp.bfloat16), sel,
                              (((1,), (1,)), ((), ())),
                              preferred_element_type=jnp.float32)
    x = x_ref[...].astype(jnp.bfloat16)
    acc = jax.lax.dot_general(x, w1s, (((1,), (1,)), ((), ())),
                              preferred_element_type=jnp.float32)
    out_ref[...] = jnp.maximum(acc + b1s, 0.0).astype(jnp.bfloat16)


def _out_kernel(h_ref, w2_ref, sel_ref, b2_ref, out_ref):
    w2 = w2_ref[...].astype(jnp.bfloat16)
    w2s = jax.lax.dot_general(w2, sel_ref[...], (((1,), (1,)), ((), ())),
                              preferred_element_type=jnp.float32)
    w2s = w2s.astype(jnp.bfloat16)
    acc = jax.lax.dot_general(h_ref[...], w2s, (((1,), (1,)), ((), ())),
                              preferred_element_type=jnp.float32)
    out_ref[...] = acc + b2_ref[...]


@jax.jit
def kernel(X, W1, b1, Hw, W2, b2):
    hw_t = Hw.reshape(L * K, INPUT_SIZE).T
    g = jnp.asarray(_G_NP)

    cq, cw = pl.pallas_call(
        _hash_kernel,
        out_shape=[jax.ShapeDtypeStruct((BATCH, L), jnp.int32),
                   jax.ShapeDtypeStruct((L, HIDDEN_SIZE), jnp.int32)],
    )(X, W1, hw_t, g)

    cw_tiled = cw.reshape(L, SC_SUBCORES, HPW).transpose(1, 0, 2).reshape(-1)
    counts2 = _make_sc_counts()(cq.reshape(BATCH * L), cw_tiled)

    b1r = b1.reshape(1, HIDDEN_SIZE)
    hlog_s, sel = pl.pallas_call(
        _hidden_kernel,
        out_shape=[jax.ShapeDtypeStruct((BATCH, NUM_SAMPLED), jnp.bfloat16),
                   jax.ShapeDtypeStruct((NUM_SAMPLED, HIDDEN_SIZE), jnp.bfloat16)],
    )(counts2.reshape(SC_CORES, HIDDEN_SIZE), X, W1, b1r)

    b2r = b2.reshape(1, NUM_CLASSES)
    CB = 512
    out = pl.pallas_call(
        _out_kernel,
        grid=(NUM_CLASSES // CB,),
        in_specs=[
            pl.BlockSpec((BATCH, NUM_SAMPLED), lambda i: (0, 0)),
            pl.BlockSpec((CB, HIDDEN_SIZE), lambda i: (i, 0)),
            pl.BlockSpec((NUM_SAMPLED, HIDDEN_SIZE), lambda i: (0, 0)),
            pl.BlockSpec((1, CB), lambda i: (0, i)),
        ],
        out_specs=pl.BlockSpec((BATCH, CB), lambda i: (0, i)),
        out_shape=jax.ShapeDtypeStruct((BATCH, NUM_CLASSES), jnp.float32),
    )(hlog_s, W2, sel, b2r)
    return out
